# baseline (speedup 1.0000x reference)
import functools

import jax
import jax.numpy as jnp
from jax import lax
from jax.experimental import pallas as pl
from jax.experimental.pallas import tpu as pltpu
from jax.experimental.pallas import tpu_sc as plsc

BATCH = 4096
SEQ = 50
EMBED_DIM = 128

NC = 2
NS = 16
NW = NC * NS
B_PER_W = BATCH // NW
ROWS_PER_CHUNK = 4
TOK_PER_CHUNK = ROWS_PER_CHUNK * SEQ
PAD_TOK = -(-TOK_PER_CHUNK // 8) * 8
N_CHUNKS = B_PER_W // ROWS_PER_CHUNK
NBUF = 2
LANES = 16
NVREG = EMBED_DIM // LANES


def _worker_id():
    return lax.axis_index("s") * NC + lax.axis_index("c")


def _make_sc_kernel():
    mesh = plsc.VectorSubcoreMesh(core_axis_name="c", subcore_axis_name="s",
                                  num_cores=NC)

    @functools.partial(
        pl.kernel,
        mesh=mesh,
        out_type=jax.ShapeDtypeStruct((BATCH, EMBED_DIM), jnp.float32),
        scratch_types=(
            [pltpu.VMEM((N_CHUNKS * PAD_TOK,), jnp.int32)]
            + [pltpu.VMEM((TOK_PER_CHUNK, EMBED_DIM), jnp.float32)
               for _ in range(NBUF)]
            + [pltpu.VMEM((B_PER_W, EMBED_DIM), jnp.float32)]
            + [pltpu.SemaphoreType.DMA for _ in range(NBUF)]
        ),
    )
    def sc_kernel(table_hbm, idx_hbm, out_hbm, idx_v, *rest):
        bufs = rest[:NBUF]
        out_v = rest[NBUF]
        sems = rest[NBUF + 1:]
        wid = _worker_id()
        inv = jnp.float32(1.0 / SEQ)

        pltpu.sync_copy(idx_hbm.at[wid], idx_v)

        def start_gather(c, b):
            pltpu.make_async_copy(
                table_hbm.at[idx_v.at[pl.ds(c * PAD_TOK, TOK_PER_CHUNK)]],
                bufs[b], sems[b]).start()

        def wait_gather(c, b):
            pltpu.make_async_copy(
                table_hbm.at[idx_v.at[pl.ds(c * PAD_TOK, TOK_PER_CHUNK)]],
                bufs[b], sems[b]).wait()

        def compute_chunk(c, b):
            bref = bufs[b]
            for r in range(ROWS_PER_CHUNK):
                base = r * SEQ

                def tbody(t, acc):
                    row = base + t
                    return tuple(acc[j] + bref[row, pl.ds(LANES * j, LANES)]
                                 for j in range(NVREG))

                acc = lax.fori_loop(
                    0, SEQ, tbody,
                    tuple(jnp.zeros((LANES,), jnp.float32)
                          for _ in range(NVREG)),
                    unroll=5)
                orow = ROWS_PER_CHUNK * c + r
                for j in range(NVREG):
                    out_v[orow, pl.ds(LANES * j, LANES)] = acc[j] * inv

        for b in range(NBUF):
            start_gather(b, b)

        def cbody(cg, carry):
            for b in range(NBUF):
                c = cg * NBUF + b
                wait_gather(c, b)
                compute_chunk(c, b)
                start_gather(c + NBUF, b)
            return carry

        lax.fori_loop(0, (N_CHUNKS - NBUF) // NBUF, cbody, 0)

        for b in range(NBUF):
            c = N_CHUNKS - NBUF + b
            wait_gather(c, b)
            compute_chunk(c, b)

        pltpu.sync_copy(out_v, out_hbm.at[pl.ds(wid * B_PER_W, B_PER_W)])

    return sc_kernel


_SC_KERNEL = _make_sc_kernel()


def kernel(input_ids, embed_table):
    ids = input_ids.astype(jnp.int32).reshape(NW, N_CHUNKS, TOK_PER_CHUNK)
    ids = jnp.pad(ids, ((0, 0), (0, 0), (0, PAD_TOK - TOK_PER_CHUNK)))
    return _SC_KERNEL(embed_table, ids.reshape(NW, N_CHUNKS * PAD_TOK))


# speedup vs baseline: 1.1661x; 1.1661x over previous
import functools

import jax
import jax.numpy as jnp
from jax import lax
from jax.experimental import pallas as pl
from jax.experimental.pallas import tpu as pltpu
from jax.experimental.pallas import tpu_sc as plsc

BATCH = 4096
SEQ = 50
EMBED_DIM = 128

NC = 2
NS = 16
NW = NC * NS
B_PER_W = BATCH // NW
ROWS_PER_CHUNK = 2
TOK_PER_CHUNK = ROWS_PER_CHUNK * SEQ
PAD_TOK = -(-TOK_PER_CHUNK // 8) * 8
N_CHUNKS = B_PER_W // ROWS_PER_CHUNK
NBUF = 4
LANES = 16
NVREG = EMBED_DIM // LANES


def _worker_id():
    return lax.axis_index("s") * NC + lax.axis_index("c")


def _make_sc_kernel():
    mesh = plsc.VectorSubcoreMesh(core_axis_name="c", subcore_axis_name="s",
                                  num_cores=NC)

    @functools.partial(
        pl.kernel,
        mesh=mesh,
        out_type=jax.ShapeDtypeStruct((BATCH, EMBED_DIM), jnp.float32),
        scratch_types=(
            [pltpu.VMEM((N_CHUNKS * PAD_TOK,), jnp.int32)]
            + [pltpu.VMEM((TOK_PER_CHUNK, EMBED_DIM), jnp.float32)
               for _ in range(NBUF)]
            + [pltpu.VMEM((B_PER_W, EMBED_DIM), jnp.float32)]
            + [pltpu.SemaphoreType.DMA for _ in range(NBUF)]
        ),
    )
    def sc_kernel(table_hbm, idx_hbm, out_hbm, idx_v, *rest):
        bufs = rest[:NBUF]
        out_v = rest[NBUF]
        sems = rest[NBUF + 1:]
        wid = _worker_id()
        inv = jnp.float32(1.0 / SEQ)

        pltpu.sync_copy(idx_hbm.at[wid], idx_v)

        def start_gather(c, b):
            pltpu.make_async_copy(
                table_hbm.at[idx_v.at[pl.ds(c * PAD_TOK, TOK_PER_CHUNK)]],
                bufs[b], sems[b]).start()

        def wait_gather(c, b):
            pltpu.make_async_copy(
                table_hbm.at[idx_v.at[pl.ds(c * PAD_TOK, TOK_PER_CHUNK)]],
                bufs[b], sems[b]).wait()

        def compute_chunk(c, b):
            bref = bufs[b]
            for r in range(ROWS_PER_CHUNK):
                base = r * SEQ

                def tbody(t, acc):
                    row = base + t
                    return tuple(acc[j] + bref[row, pl.ds(LANES * j, LANES)]
                                 for j in range(NVREG))

                acc = lax.fori_loop(
                    0, SEQ, tbody,
                    tuple(jnp.zeros((LANES,), jnp.float32)
                          for _ in range(NVREG)),
                    unroll=5)
                orow = ROWS_PER_CHUNK * c + r
                for j in range(NVREG):
                    out_v[orow, pl.ds(LANES * j, LANES)] = acc[j] * inv

        for b in range(NBUF):
            start_gather(b, b)

        def cbody(cg, carry):
            for b in range(NBUF):
                c = cg * NBUF + b
                wait_gather(c, b)
                compute_chunk(c, b)
                start_gather(c + NBUF, b)
            return carry

        lax.fori_loop(0, (N_CHUNKS - NBUF) // NBUF, cbody, 0)

        for b in range(NBUF):
            c = N_CHUNKS - NBUF + b
            wait_gather(c, b)
            compute_chunk(c, b)

        pltpu.sync_copy(out_v, out_hbm.at[pl.ds(wid * B_PER_W, B_PER_W)])

    return sc_kernel


_SC_KERNEL = _make_sc_kernel()


def kernel(input_ids, embed_table):
    ids = input_ids.astype(jnp.int32).reshape(NW, N_CHUNKS, TOK_PER_CHUNK)
    ids = jnp.pad(ids, ((0, 0), (0, 0), (0, PAD_TOK - TOK_PER_CHUNK)))
    return _SC_KERNEL(embed_table, ids.reshape(NW, N_CHUNKS * PAD_TOK))


# speedup vs baseline: 1.2006x; 1.0296x over previous
import functools

import jax
import jax.numpy as jnp
from jax import lax
from jax.experimental import pallas as pl
from jax.experimental.pallas import tpu as pltpu
from jax.experimental.pallas import tpu_sc as plsc

BATCH = 4096
SEQ = 50
EMBED_DIM = 128

NC = 2
NS = 16
NW = NC * NS
B_PER_W = BATCH // NW
ROWS_PER_CHUNK = 1
TOK_PER_CHUNK = ROWS_PER_CHUNK * SEQ
PAD_TOK = -(-TOK_PER_CHUNK // 8) * 8
N_CHUNKS = B_PER_W // ROWS_PER_CHUNK
NBUF = 8
LANES = 16
NVREG = EMBED_DIM // LANES


def _worker_id():
    return lax.axis_index("s") * NC + lax.axis_index("c")


def _make_sc_kernel():
    mesh = plsc.VectorSubcoreMesh(core_axis_name="c", subcore_axis_name="s",
                                  num_cores=NC)

    @functools.partial(
        pl.kernel,
        mesh=mesh,
        out_type=jax.ShapeDtypeStruct((BATCH, EMBED_DIM), jnp.float32),
        scratch_types=(
            [pltpu.VMEM((N_CHUNKS * PAD_TOK,), jnp.int32)]
            + [pltpu.VMEM((TOK_PER_CHUNK, EMBED_DIM), jnp.float32)
               for _ in range(NBUF)]
            + [pltpu.VMEM((B_PER_W, EMBED_DIM), jnp.float32)]
            + [pltpu.SemaphoreType.DMA for _ in range(NBUF)]
        ),
    )
    def sc_kernel(table_hbm, idx_hbm, out_hbm, idx_v, *rest):
        bufs = rest[:NBUF]
        out_v = rest[NBUF]
        sems = rest[NBUF + 1:]
        wid = _worker_id()
        inv = jnp.float32(1.0 / SEQ)

        pltpu.sync_copy(idx_hbm.at[wid], idx_v)

        def start_gather(c, b):
            pltpu.make_async_copy(
                table_hbm.at[idx_v.at[pl.ds(c * PAD_TOK, TOK_PER_CHUNK)]],
                bufs[b], sems[b]).start()

        def wait_gather(c, b):
            pltpu.make_async_copy(
                table_hbm.at[idx_v.at[pl.ds(c * PAD_TOK, TOK_PER_CHUNK)]],
                bufs[b], sems[b]).wait()

        def compute_chunk(c, b):
            bref = bufs[b]
            for r in range(ROWS_PER_CHUNK):
                base = r * SEQ

                def tbody(t, acc):
                    row = base + t
                    return tuple(acc[j] + bref[row, pl.ds(LANES * j, LANES)]
                                 for j in range(NVREG))

                acc = lax.fori_loop(
                    0, SEQ, tbody,
                    tuple(jnp.zeros((LANES,), jnp.float32)
                          for _ in range(NVREG)),
                    unroll=5)
                orow = ROWS_PER_CHUNK * c + r
                for j in range(NVREG):
                    out_v[orow, pl.ds(LANES * j, LANES)] = acc[j] * inv

        for b in range(NBUF):
            start_gather(b, b)

        def cbody(cg, carry):
            for b in range(NBUF):
                c = cg * NBUF + b
                wait_gather(c, b)
                compute_chunk(c, b)
                start_gather(c + NBUF, b)
            return carry

        lax.fori_loop(0, (N_CHUNKS - NBUF) // NBUF, cbody, 0)

        for b in range(NBUF):
            c = N_CHUNKS - NBUF + b
            wait_gather(c, b)
            compute_chunk(c, b)

        pltpu.sync_copy(out_v, out_hbm.at[pl.ds(wid * B_PER_W, B_PER_W)])

    return sc_kernel


_SC_KERNEL = _make_sc_kernel()


def kernel(input_ids, embed_table):
    ids = input_ids.astype(jnp.int32).reshape(NW, N_CHUNKS, TOK_PER_CHUNK)
    ids = jnp.pad(ids, ((0, 0), (0, 0), (0, PAD_TOK - TOK_PER_CHUNK)))
    return _SC_KERNEL(embed_table, ids.reshape(NW, N_CHUNKS * PAD_TOK))
